# baseline (speedup 1.0000x reference)
import functools
import jax
import jax.numpy as jnp
from jax import lax
from jax.experimental import pallas as pl
from jax.experimental.pallas import tpu as pltpu
from jax.experimental.pallas import tpu_sc as plsc

NUM_EMBEDDINGS = 1000000
EMBEDDING_DIM = 64
SEQ_LEN = 200
BATCH = 4096

_B = SEQ_LEN * BATCH
_NC, _NS = 2, 16
_NW = _NC * _NS
_BPW = _B // _NW
_C = 512
_NCHUNK = _BPW // _C


def _make_gather():
    mesh = plsc.VectorSubcoreMesh(core_axis_name="c", subcore_axis_name="s")

    @functools.partial(
        pl.kernel,
        mesh=mesh,
        out_type=jax.ShapeDtypeStruct((_B, EMBEDDING_DIM), jnp.float32),
        compiler_params=pltpu.CompilerParams(use_tc_tiling_on_sc=False),
        scratch_types=[
            pltpu.VMEM((_C,), jnp.int32),
            pltpu.VMEM((_C, EMBEDDING_DIM), jnp.float32),
            pltpu.SemaphoreType.DMA,
        ],
    )
    def gather_kernel(table_hbm, idx_hbm, out_hbm, idx_v, rows_v, sem):
        wid = lax.axis_index("s") * _NC + lax.axis_index("c")
        base = wid * _BPW

        def body(i, _):
            off = base + i * _C
            pltpu.sync_copy(idx_hbm.at[pl.ds(off, _C)], idx_v)
            pltpu.async_copy(table_hbm.at[idx_v], rows_v, sem).wait()
            pltpu.sync_copy(rows_v, out_hbm.at[pl.ds(off, _C)])
            return _

        lax.fori_loop(0, _NCHUNK, body, 0)

    return gather_kernel


_gather = _make_gather()


def kernel(input, weight):
    flat_idx = input.reshape(-1)
    rows = _gather(weight, flat_idx)
    return rows.reshape(input.shape + (weight.shape[1],))


# speedup vs baseline: 1.0292x; 1.0292x over previous
import functools
import jax
import jax.numpy as jnp
from jax import lax
from jax.experimental import pallas as pl
from jax.experimental.pallas import tpu as pltpu
from jax.experimental.pallas import tpu_sc as plsc

NUM_EMBEDDINGS = 1000000
EMBEDDING_DIM = 64
SEQ_LEN = 200
BATCH = 4096

_B = SEQ_LEN * BATCH
_NC, _NS = 2, 16
_NW = _NC * _NS
_BPW = _B // _NW
_C = 800
_NCHUNK = _BPW // _C


def _make_gather():
    mesh = plsc.VectorSubcoreMesh(core_axis_name="c", subcore_axis_name="s")

    @functools.partial(
        pl.kernel,
        mesh=mesh,
        out_type=jax.ShapeDtypeStruct((_B, EMBEDDING_DIM), jnp.float32),
        compiler_params=pltpu.CompilerParams(use_tc_tiling_on_sc=False),
        scratch_types=[
            pltpu.VMEM((2, _C), jnp.int32),
            pltpu.VMEM((2, _C, EMBEDDING_DIM), jnp.float32),
            pltpu.SemaphoreType.DMA,
            pltpu.SemaphoreType.DMA,
            pltpu.SemaphoreType.DMA,
            pltpu.SemaphoreType.DMA,
        ],
    )
    def gather_kernel(table_hbm, idx_hbm, out_hbm, idx_v, rows_v, g0, g1,
                      s0, s1):
        wid = lax.axis_index("s") * _NC + lax.axis_index("c")
        base = wid * _BPW
        gsem = (g0, g1)
        ssem = (s0, s1)

        def load_idx_and_gather(b, off):
            pltpu.sync_copy(idx_hbm.at[pl.ds(off, _C)], idx_v.at[b])
            pltpu.async_copy(table_hbm.at[idx_v.at[b]], rows_v.at[b], gsem[b])

        def wait_gather(b):
            pltpu.make_async_copy(
                table_hbm.at[idx_v.at[b]], rows_v.at[b], gsem[b]).wait()

        def start_store(b, off):
            pltpu.async_copy(rows_v.at[b], out_hbm.at[pl.ds(off, _C)], ssem[b])

        def wait_store(b):
            pltpu.make_async_copy(
                rows_v.at[b], out_hbm.at[pl.ds(base, _C)], ssem[b]).wait()

        load_idx_and_gather(0, base)

        def body(s, carry):
            off0 = base + (2 * s) * _C
            off1 = off0 + _C

            wait_gather(0)

            @pl.when(s > 0)
            def _():
                wait_store(1)

            load_idx_and_gather(1, off1)
            start_store(0, off0)

            wait_gather(1)
            wait_store(0)

            @pl.when(s + 1 < _NCHUNK // 2)
            def _():
                load_idx_and_gather(0, off1 + _C)

            start_store(1, off1)
            return carry

        lax.fori_loop(0, _NCHUNK // 2, body, 0)
        wait_store(1)

    return gather_kernel


_gather = _make_gather()


def kernel(input, weight):
    flat_idx = input.reshape(-1)
    rows = _gather(weight, flat_idx)
    return rows.reshape(input.shape + (weight.shape[1],))
